# baseline (speedup 1.0000x reference)
import jax
import jax.numpy as jnp
from jax import lax
from jax.experimental import pallas as pl

ALPHA = 16.0


def _fused_body(ids_ref, x_ref, w_ref, b_ref, af_ref, bf_ref, o_ref, *, r, er):
    x = x_ref[...]
    base = jnp.dot(x, w_ref[...], preferred_element_type=jnp.float32)
    lr = jnp.dot(x, af_ref[...], preferred_element_type=jnp.float32)
    ids = ids_ref[...]
    col_expert = lax.broadcasted_iota(jnp.int32, lr.shape, 1) // r
    mask = (col_expert == ids).astype(jnp.float32)
    delta = jnp.dot(lr * mask, bf_ref[...], preferred_element_type=jnp.float32)
    o_ref[...] = base + b_ref[...] + delta * (ALPHA / r)


def kernel(hidden_states, base_weight, base_bias, lora_a, lora_b, adapter_ids):
    n, d_in = hidden_states.shape
    d_out = base_weight.shape[1]
    e, _, r = lora_a.shape
    er = e * r

    a_flat = jnp.transpose(lora_a, (1, 0, 2)).reshape(d_in, er)
    b_flat = lora_b.reshape(er, d_out)
    ids2d = adapter_ids.astype(jnp.int32).reshape(n, 1)
    bias2d = base_bias.reshape(1, d_out)

    tm = 2048
    grid = (n // tm,)

    import functools
    body = functools.partial(_fused_body, r=r, er=er)
    return pl.pallas_call(
        body,
        grid=grid,
        in_specs=[
            pl.BlockSpec((tm, 1), lambda i: (i, 0)),
            pl.BlockSpec((tm, d_in), lambda i: (i, 0)),
            pl.BlockSpec((d_in, d_out), lambda i: (0, 0)),
            pl.BlockSpec((1, d_out), lambda i: (0, 0)),
            pl.BlockSpec((d_in, er), lambda i: (0, 0)),
            pl.BlockSpec((er, d_out), lambda i: (0, 0)),
        ],
        out_specs=pl.BlockSpec((tm, d_out), lambda i: (i, 0)),
        out_shape=jax.ShapeDtypeStruct((n, d_out), jnp.float32),
    )(ids2d, hidden_states, base_weight, bias2d, a_flat, b_flat)


# speedup vs baseline: 1.0114x; 1.0114x over previous
import jax
import jax.numpy as jnp
from jax import lax
from jax.experimental import pallas as pl
from jax.experimental.pallas import tpu as pltpu

ALPHA = 16.0


def _fused_body(ids_ref, x_ref, w_ref, b_ref, af_ref, bf_ref, o_ref, *, r, er):
    x = x_ref[...]
    base = jnp.dot(x, w_ref[...], preferred_element_type=jnp.float32)
    lr = jnp.dot(x, af_ref[...], preferred_element_type=jnp.float32)
    ids = ids_ref[...]
    col_expert = lax.broadcasted_iota(jnp.int32, lr.shape, 1) // r
    mask = (col_expert == ids).astype(jnp.float32)
    delta = jnp.dot(lr * mask, bf_ref[...], preferred_element_type=jnp.float32)
    o_ref[...] = base + b_ref[...] + delta * (ALPHA / r)


def kernel(hidden_states, base_weight, base_bias, lora_a, lora_b, adapter_ids):
    n, d_in = hidden_states.shape
    d_out = base_weight.shape[1]
    e, _, r = lora_a.shape
    er = e * r

    a_flat = jnp.transpose(lora_a, (1, 0, 2)).reshape(d_in, er)
    b_flat = lora_b.reshape(er, d_out)
    ids2d = adapter_ids.astype(jnp.int32).reshape(n, 1)
    bias2d = base_bias.reshape(1, d_out)

    tm = 1024
    grid = (n // tm,)

    import functools
    body = functools.partial(_fused_body, r=r, er=er)
    return pl.pallas_call(
        body,
        grid=grid,
        in_specs=[
            pl.BlockSpec((tm, 1), lambda i: (i, 0)),
            pl.BlockSpec((tm, d_in), lambda i: (i, 0)),
            pl.BlockSpec((d_in, d_out), lambda i: (0, 0)),
            pl.BlockSpec((1, d_out), lambda i: (0, 0)),
            pl.BlockSpec((d_in, er), lambda i: (0, 0)),
            pl.BlockSpec((er, d_out), lambda i: (0, 0)),
        ],
        out_specs=pl.BlockSpec((tm, d_out), lambda i: (i, 0)),
        out_shape=jax.ShapeDtypeStruct((n, d_out), jnp.float32),
        compiler_params=pltpu.CompilerParams(
            dimension_semantics=("parallel",),
        ),
    )(ids2d, hidden_states, base_weight, bias2d, a_flat, b_flat)


# speedup vs baseline: 1.0115x; 1.0001x over previous
import jax
import jax.numpy as jnp
from jax import lax
from jax.experimental import pallas as pl
from jax.experimental.pallas import tpu as pltpu

ALPHA = 16.0


def _fused_body(ids_ref, x_ref, w_ref, b_ref, af_ref, bf_ref, o_ref, *, r, er):
    x = x_ref[...].astype(jnp.bfloat16)
    w = w_ref[...].astype(jnp.bfloat16)
    base = jnp.dot(x, w, preferred_element_type=jnp.float32)
    lr = jnp.dot(x, af_ref[...].astype(jnp.bfloat16),
                 preferred_element_type=jnp.float32)
    ids = ids_ref[...]
    col_expert = lax.broadcasted_iota(jnp.int32, lr.shape, 1) // r
    mask = (col_expert == ids).astype(jnp.float32)
    delta = jnp.dot((lr * mask).astype(jnp.bfloat16),
                    bf_ref[...].astype(jnp.bfloat16),
                    preferred_element_type=jnp.float32)
    o_ref[...] = base + b_ref[...] + delta * (ALPHA / r)


def kernel(hidden_states, base_weight, base_bias, lora_a, lora_b, adapter_ids):
    n, d_in = hidden_states.shape
    d_out = base_weight.shape[1]
    e, _, r = lora_a.shape
    er = e * r

    a_flat = jnp.transpose(lora_a, (1, 0, 2)).reshape(d_in, er)
    b_flat = lora_b.reshape(er, d_out)
    ids2d = adapter_ids.astype(jnp.int32).reshape(n, 1)
    bias2d = base_bias.reshape(1, d_out)

    tm = 1024
    grid = (n // tm,)

    import functools
    body = functools.partial(_fused_body, r=r, er=er)
    return pl.pallas_call(
        body,
        grid=grid,
        in_specs=[
            pl.BlockSpec((tm, 1), lambda i: (i, 0)),
            pl.BlockSpec((tm, d_in), lambda i: (i, 0)),
            pl.BlockSpec((d_in, d_out), lambda i: (0, 0)),
            pl.BlockSpec((1, d_out), lambda i: (0, 0)),
            pl.BlockSpec((d_in, er), lambda i: (0, 0)),
            pl.BlockSpec((er, d_out), lambda i: (0, 0)),
        ],
        out_specs=pl.BlockSpec((tm, d_out), lambda i: (i, 0)),
        out_shape=jax.ShapeDtypeStruct((n, d_out), jnp.float32),
        compiler_params=pltpu.CompilerParams(
            dimension_semantics=("parallel",),
        ),
    )(ids2d, hidden_states, base_weight, bias2d, a_flat, b_flat)


# speedup vs baseline: 1.0482x; 1.0363x over previous
import functools

import jax
import jax.numpy as jnp
from jax import lax
from jax.experimental import pallas as pl
from jax.experimental.pallas import tpu as pltpu

ALPHA = 16.0


def _fused_body(ids_ref, x_ref, wa_ref, b_ref, bf_ref, o_ref, *, r, d_out):
    x = x_ref[...]
    y = jnp.dot(x, wa_ref[...], preferred_element_type=jnp.float32)
    base = y[:, :d_out]
    lr = y[:, d_out:]
    ids = ids_ref[...]
    col_expert = lax.broadcasted_iota(jnp.int32, lr.shape, 1) // r
    mask = (col_expert == ids).astype(jnp.float32)
    delta = jnp.dot(lr * mask, bf_ref[...], preferred_element_type=jnp.float32)
    o_ref[...] = base + b_ref[...] + delta * (ALPHA / r)


def kernel(hidden_states, base_weight, base_bias, lora_a, lora_b, adapter_ids):
    n, d_in = hidden_states.shape
    d_out = base_weight.shape[1]
    e, _, r = lora_a.shape
    er = e * r

    a_flat = jnp.transpose(lora_a, (1, 0, 2)).reshape(d_in, er)
    wa = jnp.concatenate([base_weight, a_flat], axis=1)
    b_flat = lora_b.reshape(er, d_out)
    ids2d = adapter_ids.astype(jnp.int32).reshape(n, 1)
    bias2d = base_bias.reshape(1, d_out)

    tm = 1024
    grid = (n // tm,)

    body = functools.partial(_fused_body, r=r, d_out=d_out)
    return pl.pallas_call(
        body,
        grid=grid,
        in_specs=[
            pl.BlockSpec((tm, 1), lambda i: (i, 0)),
            pl.BlockSpec((tm, d_in), lambda i: (i, 0)),
            pl.BlockSpec((d_in, d_out + er), lambda i: (0, 0)),
            pl.BlockSpec((1, d_out), lambda i: (0, 0)),
            pl.BlockSpec((er, d_out), lambda i: (0, 0)),
        ],
        out_specs=pl.BlockSpec((tm, d_out), lambda i: (i, 0)),
        out_shape=jax.ShapeDtypeStruct((n, d_out), jnp.float32),
        compiler_params=pltpu.CompilerParams(
            dimension_semantics=("parallel",),
        ),
    )(ids2d, hidden_states, wa, bias2d, b_flat)


# speedup vs baseline: 1.2068x; 1.1513x over previous
import functools

import jax
import jax.numpy as jnp
from jax import lax
from jax.experimental import pallas as pl
from jax.experimental.pallas import tpu as pltpu

ALPHA = 16.0


def _fused_body(ids_ref, x_ref, wa_ref, b_ref, bf_ref, o_ref, *, r, d_out):
    x = x_ref[...]
    y = jnp.dot(x, wa_ref[...], preferred_element_type=jnp.float32)
    base = y[:, :d_out]
    lr = y[:, d_out:]
    ids = ids_ref[...].reshape(lr.shape[0], 1)
    col_expert = lax.broadcasted_iota(jnp.int32, lr.shape, 1) // r
    mask = (col_expert == ids).astype(jnp.float32)
    delta = jnp.dot(lr * mask, bf_ref[...], preferred_element_type=jnp.float32)
    o_ref[...] = base + b_ref[...].reshape(1, d_out) + delta * (ALPHA / r)


def kernel(hidden_states, base_weight, base_bias, lora_a, lora_b, adapter_ids):
    n, d_in = hidden_states.shape
    d_out = base_weight.shape[1]
    e, _, r = lora_a.shape
    er = e * r

    a_flat = jnp.transpose(lora_a, (1, 0, 2)).reshape(d_in, er)
    wa = jnp.concatenate([base_weight, a_flat], axis=1)
    b_flat = lora_b.reshape(er, d_out)

    tm = 1024
    grid = (n // tm,)

    body = functools.partial(_fused_body, r=r, d_out=d_out)
    return pl.pallas_call(
        body,
        grid=grid,
        in_specs=[
            pl.BlockSpec((tm,), lambda i: (i,)),
            pl.BlockSpec((tm, d_in), lambda i: (i, 0)),
            pl.BlockSpec((d_in, d_out + er), lambda i: (0, 0)),
            pl.BlockSpec((d_out,), lambda i: (0,)),
            pl.BlockSpec((er, d_out), lambda i: (0, 0)),
        ],
        out_specs=pl.BlockSpec((tm, d_out), lambda i: (i, 0)),
        out_shape=jax.ShapeDtypeStruct((n, d_out), jnp.float32),
        compiler_params=pltpu.CompilerParams(
            dimension_semantics=("parallel",),
        ),
    )(adapter_ids.astype(jnp.int32), hidden_states, wa, base_bias, b_flat)
